# baseline (speedup 1.0000x reference)
import jax
import jax.numpy as jnp
from jax import lax
from jax.experimental import pallas as pl
from jax.experimental.pallas import tpu as pltpu
from jax.experimental.pallas import tpu_sc as plsc

B, S, D = 4096, 200, 64
N = B * S
EPS = 1e-12
NC, NS = 2, 16
NW = NC * NS
TOK_W = N // NW
CHUNK = S
NCHUNK = TOK_W // CHUNK
L = 16
K = D // L
DP = 128
H1 = 104
H2 = CHUNK - H1


def _rsqrt16(v):
  i = plsc.bitcast(v, jnp.int32)
  i = jnp.int32(0x5F3759DF) - lax.shift_right_arithmetic(i, 1)
  r = plsc.bitcast(i, jnp.float32)
  for _ in range(3):
    r = r * (1.5 - 0.5 * v * r * r)
  return r


def _body(x_hbm, ids_hbm, pos_hbm, lab_hbm, gam_hbm, bet_hbm, out_hbm,
          xb0, xb1, lb0, lb1, ob, idb0, idb1, posv, gv, bv,
          sx0, sx1, si0, si1, sg0, sg1, so):
  xb = (xb0, xb1)
  lb = (lb0, lb1)
  idb = (idb0, idb1)
  sx = (sx0, sx1)
  si = (si0, si1)
  sg = (sg0, sg1)

  wid = lax.axis_index("subcore") * NC + lax.axis_index("core")
  base0 = wid * TOK_W

  pltpu.sync_copy(pos_hbm, posv)
  pltpu.sync_copy(gam_hbm, gv)
  pltpu.sync_copy(bet_hbm, bv)
  g = [gv[pl.ds(k * L, L)] for k in range(K)]
  bt = [bv[pl.ds(k * L, L)] for k in range(K)]

  def issue_ids(c, p):
    pltpu.async_copy(ids_hbm.at[pl.ds(base0 + c * CHUNK, CHUNK)],
                     idb[p], si[p])

  def wait_ids(p):
    pltpu.make_async_copy(ids_hbm.at[pl.ds(0, CHUNK)], idb[p], si[p]).wait()

  def issue_x(c, p):
    pltpu.async_copy(x_hbm.at[pl.ds((base0 + c * CHUNK) * D, CHUNK * D)],
                     xb[p], sx[p])

  def wait_x(p):
    pltpu.make_async_copy(x_hbm.at[pl.ds(0, CHUNK * D)], xb[p], sx[p]).wait()

  def issue_gather(p):
    pltpu.async_copy(lab_hbm.at[idb[p].at[pl.ds(0, H1)]],
                     lb[p].at[pl.ds(0, H1)], sg[p])
    pltpu.async_copy(lab_hbm.at[idb[p].at[pl.ds(H1, H2)]],
                     lb[p].at[pl.ds(H1, H2)], sg[p])

  def wait_gather(p):
    pltpu.make_async_copy(lab_hbm.at[idb[p].at[pl.ds(0, H1)]],
                          lb[p].at[pl.ds(0, H1)], sg[p]).wait()
    pltpu.make_async_copy(lab_hbm.at[idb[p].at[pl.ds(H1, H2)]],
                          lb[p].at[pl.ds(H1, H2)], sg[p]).wait()

  def issue_out(c):
    pltpu.async_copy(ob, out_hbm.at[pl.ds((base0 + c * CHUNK) * D, CHUNK * D)],
                     so)

  def wait_out():
    pltpu.make_async_copy(ob, out_hbm.at[pl.ds(0, CHUNK * D)], so).wait()

  def compute(p):
    xp, lp, op = xb[p], lb[p], ob

    @plsc.parallel_loop(0, CHUNK, unroll=4)
    def _tok(t):
      tb = t * D
      e = []
      for k in range(K):
        e.append(xp[pl.ds(tb + k * L, L)] + posv[pl.ds(tb + k * L, L)]
                 + lp[t, pl.ds(k * L, L)])
      ssum = jnp.sum(e[0] + e[1] + e[2] + e[3])
      qsum = jnp.sum(e[0] * e[0] + e[1] * e[1] + e[2] * e[2] + e[3] * e[3])
      mean = ssum * (1.0 / D)
      var = qsum * (1.0 / D) - mean * mean
      r = _rsqrt16(jnp.broadcast_to(var + EPS, (L,)))
      for k in range(K):
        op[pl.ds(tb + k * L, L)] = (e[k] - mean) * r * g[k] + bt[k]

  def stage(c, p, *, first=False, no_next_gather=False, no_prefetch=False):
    q = 1 - p
    if not no_next_gather:
      wait_ids(q)
      issue_gather(q)
    wait_x(p)
    wait_gather(p)
    if not no_prefetch:
      issue_ids(c + 2, p)
    if not first:
      wait_out()
    compute(p)
    issue_out(c)
    if not no_prefetch:
      issue_x(c + 2, p)

  issue_ids(0, 0)
  issue_x(0, 0)
  issue_ids(1, 1)
  issue_x(1, 1)
  wait_ids(0)
  issue_gather(0)

  stage(0, 0, first=True)
  stage(1, 1, first=True)

  @pl.loop(1, NCHUNK // 2 - 1)
  def _pair(cc):
    stage(2 * cc, 0)
    stage(2 * cc + 1, 1)

  stage(NCHUNK - 2, 0, no_prefetch=True)
  stage(NCHUNK - 1, 1, no_next_gather=True, no_prefetch=True)
  wait_out()


@jax.jit
def kernel(input_tensor, label_ids, pos_table, label_table, ln_gamma, ln_beta):
  x2 = input_tensor.reshape(N * D)
  ids = label_ids.reshape(N).astype(jnp.int32)
  pos200 = pos_table[:S].reshape(S * D)
  labp = jnp.pad(label_table, ((0, 0), (0, DP - D)))
  mesh = plsc.VectorSubcoreMesh(core_axis_name="core",
                                subcore_axis_name="subcore")
  cp = pltpu.CompilerParams(needs_layout_passes=False)
  run = pl.kernel(
      _body,
      out_type=jax.ShapeDtypeStruct((N * D,), jnp.float32),
      mesh=mesh,
      scratch_types=[
          pltpu.VMEM((CHUNK * D,), jnp.float32),
          pltpu.VMEM((CHUNK * D,), jnp.float32),
          pltpu.VMEM((CHUNK, DP), jnp.float32),
          pltpu.VMEM((CHUNK, DP), jnp.float32),
          pltpu.VMEM((CHUNK * D,), jnp.float32),
          pltpu.VMEM((CHUNK,), jnp.int32),
          pltpu.VMEM((CHUNK,), jnp.int32),
          pltpu.VMEM((S * D,), jnp.float32),
          pltpu.VMEM((D,), jnp.float32),
          pltpu.VMEM((D,), jnp.float32),
          pltpu.SemaphoreType.DMA,
          pltpu.SemaphoreType.DMA,
          pltpu.SemaphoreType.DMA,
          pltpu.SemaphoreType.DMA,
          pltpu.SemaphoreType.DMA,
          pltpu.SemaphoreType.DMA,
          pltpu.SemaphoreType.DMA,
      ],
      compiler_params=cp,
  )
  out = run(x2, ids, pos200, labp, ln_gamma, ln_beta)
  return out.reshape(B, S, D)


# speedup vs baseline: 1.9259x; 1.9259x over previous
import jax
import jax.numpy as jnp
from jax import lax
from jax.experimental import pallas as pl
from jax.experimental.pallas import tpu as pltpu
from jax.experimental.pallas import tpu_sc as plsc

B, S, D = 4096, 200, 64
N = B * S
EPS = 1e-12
NC, NS = 2, 16
NW = NC * NS
TOK_W = N // NW
CHUNK = S
NCHUNK = TOK_W // CHUNK
L = 16
K = D // L
NLAB = 26
IDPAD = CHUNK + L


def _rsqrt16(v):
  i = plsc.bitcast(v, jnp.int32)
  i = jnp.int32(0x5F3759DF) - lax.shift_right_arithmetic(i, 1)
  r = plsc.bitcast(i, jnp.float32)
  for _ in range(3):
    r = r * (1.5 - 0.5 * v * r * r)
  return r


def _body(x_hbm, ids_hbm, pos_hbm, lab_hbm, gam_hbm, bet_hbm, out_hbm,
          xb0, xb1, ob, idb0, idb1, posv, tabv, gv, bv,
          sx0, sx1, si0, si1, so):
  xb = (xb0, xb1)
  idb = (idb0, idb1)
  sx = (sx0, sx1)
  si = (si0, si1)

  wid = lax.axis_index("subcore") * NC + lax.axis_index("core")
  base0 = wid * TOK_W

  pltpu.sync_copy(pos_hbm, posv)
  pltpu.sync_copy(lab_hbm, tabv)
  pltpu.sync_copy(gam_hbm, gv)
  pltpu.sync_copy(bet_hbm, bv)
  g = [gv[pl.ds(k * L, L)] for k in range(K)]
  bt = [bv[pl.ds(k * L, L)] for k in range(K)]

  def issue_ids(c, p):
    pltpu.async_copy(ids_hbm.at[pl.ds(base0 + c * CHUNK, CHUNK)],
                     idb[p].at[pl.ds(0, CHUNK)], si[p])

  def wait_ids(p):
    pltpu.make_async_copy(ids_hbm.at[pl.ds(0, CHUNK)],
                          idb[p].at[pl.ds(0, CHUNK)], si[p]).wait()

  def issue_x(c, p):
    pltpu.async_copy(x_hbm.at[pl.ds((base0 + c * CHUNK) * D, CHUNK * D)],
                     xb[p], sx[p])

  def wait_x(p):
    pltpu.make_async_copy(x_hbm.at[pl.ds(0, CHUNK * D)], xb[p], sx[p]).wait()

  def issue_out(c):
    pltpu.async_copy(ob, out_hbm.at[pl.ds((base0 + c * CHUNK) * D, CHUNK * D)],
                     so)

  def wait_out():
    pltpu.make_async_copy(ob, out_hbm.at[pl.ds(0, CHUNK * D)], so).wait()

  def compute(p):
    xp, ip, op = xb[p], idb[p], ob

    idx15 = jnp.full((L,), L - 1, jnp.int32)
    idx0 = jnp.full((L,), 0, jnp.int32)
    iota = lax.iota(jnp.int32, L)
    ck = [iota + k * L for k in range(K)]

    @plsc.parallel_loop(0, CHUNK, unroll=4)
    def _tok(t):
      tb = t * D
      idv = ip[pl.ds(t, L)]
      ibase = lax.shift_left(jnp.take(idv, idx0), 6)
      e = []
      for k in range(K):
        e.append(xp[pl.ds(tb + k * L, L)] + posv[pl.ds(tb + k * L, L)]
                 + plsc.load_gather(tabv, [ibase + ck[k]]))
      ssum = jnp.take(plsc.cumsum(e[0] + e[1] + e[2] + e[3]), idx15)
      qsum = jnp.take(plsc.cumsum(e[0] * e[0] + e[1] * e[1]
                                  + e[2] * e[2] + e[3] * e[3]), idx15)
      mean = ssum * (1.0 / D)
      var = qsum * (1.0 / D) - mean * mean
      r = _rsqrt16(var + EPS)
      for k in range(K):
        op[pl.ds(tb + k * L, L)] = (e[k] - mean) * r * g[k] + bt[k]

  def stage(c, p, *, first=False, no_prefetch=False):
    wait_x(p)
    wait_ids(p)
    if not first:
      wait_out()
    compute(p)
    issue_out(c)
    if not no_prefetch:
      issue_ids(c + 2, p)
      issue_x(c + 2, p)

  issue_ids(0, 0)
  issue_x(0, 0)
  issue_ids(1, 1)
  issue_x(1, 1)

  stage(0, 0, first=True)
  stage(1, 1)

  @pl.loop(1, NCHUNK // 2 - 1)
  def _pair(cc):
    stage(2 * cc, 0)
    stage(2 * cc + 1, 1)

  stage(NCHUNK - 2, 0, no_prefetch=True)
  stage(NCHUNK - 1, 1, no_prefetch=True)
  wait_out()


@jax.jit
def kernel(input_tensor, label_ids, pos_table, label_table, ln_gamma, ln_beta):
  x2 = input_tensor.reshape(N * D)
  ids = label_ids.reshape(N).astype(jnp.int32)
  pos200 = pos_table[:S].reshape(S * D)
  labf = label_table.reshape(NLAB * D)
  mesh = plsc.VectorSubcoreMesh(core_axis_name="core",
                                subcore_axis_name="subcore")
  cp = pltpu.CompilerParams(needs_layout_passes=False)
  run = pl.kernel(
      _body,
      out_type=jax.ShapeDtypeStruct((N * D,), jnp.float32),
      mesh=mesh,
      scratch_types=[
          pltpu.VMEM((CHUNK * D,), jnp.float32),
          pltpu.VMEM((CHUNK * D,), jnp.float32),
          pltpu.VMEM((CHUNK * D,), jnp.float32),
          pltpu.VMEM((IDPAD,), jnp.int32),
          pltpu.VMEM((IDPAD,), jnp.int32),
          pltpu.VMEM((S * D,), jnp.float32),
          pltpu.VMEM((NLAB * D,), jnp.float32),
          pltpu.VMEM((D,), jnp.float32),
          pltpu.VMEM((D,), jnp.float32),
          pltpu.SemaphoreType.DMA,
          pltpu.SemaphoreType.DMA,
          pltpu.SemaphoreType.DMA,
          pltpu.SemaphoreType.DMA,
          pltpu.SemaphoreType.DMA,
      ],
      compiler_params=cp,
  )
  out = run(x2, ids, pos200, labf, ln_gamma, ln_beta)
  return out.reshape(B, S, D)


# speedup vs baseline: 2.4624x; 1.2786x over previous
import jax
import jax.numpy as jnp
from jax import lax
from jax.experimental import pallas as pl
from jax.experimental.pallas import tpu as pltpu
from jax.experimental.pallas import tpu_sc as plsc

B, S, D = 4096, 200, 64
N = B * S
EPS = 1e-12
NC, NS = 2, 16
NW = NC * NS
ROWS_W = B // NW
CHUNK = S
L = 16
K = D // L
NLAB = 26
IDPAD = CHUNK + L


def _rsqrt16(v):
  i = plsc.bitcast(v, jnp.int32)
  i = jnp.int32(0x5F3759DF) - lax.shift_right_arithmetic(i, 1)
  r = plsc.bitcast(i, jnp.float32)
  for _ in range(3):
    r = r * (1.5 - 0.5 * v * r * r)
  return r


def _body(x_hbm, ids_hbm, pos_hbm, lab_hbm, gam_hbm, bet_hbm, out_hbm,
          xb0, xb1, ob, idb0, idb1, posv, tabv, gv, bv,
          sx0, sx1, si0, si1, so):
  xb = (xb0, xb1)
  idb = (idb0, idb1)
  sx = (sx0, sx1)
  si = (si0, si1)

  wid = lax.axis_index("subcore") * NC + lax.axis_index("core")
  row0 = wid * ROWS_W

  pltpu.sync_copy(pos_hbm.at[pl.ds(0, S)], posv)
  pltpu.sync_copy(lab_hbm, tabv)
  pltpu.sync_copy(gam_hbm, gv)
  pltpu.sync_copy(bet_hbm, bv)
  g = [gv[pl.ds(k * L, L)] for k in range(K)]
  bt = [bv[pl.ds(k * L, L)] for k in range(K)]

  def issue_ids(c, p):
    pltpu.async_copy(ids_hbm.at[pl.ds((row0 + c) * CHUNK, CHUNK)],
                     idb[p].at[pl.ds(0, CHUNK)], si[p])

  def wait_ids(p):
    pltpu.make_async_copy(ids_hbm.at[pl.ds(0, CHUNK)],
                          idb[p].at[pl.ds(0, CHUNK)], si[p]).wait()

  def issue_x(c, p):
    pltpu.async_copy(x_hbm.at[row0 + c], xb[p], sx[p])

  def wait_x(p):
    pltpu.make_async_copy(x_hbm.at[0], xb[p], sx[p]).wait()

  def issue_out(c):
    pltpu.async_copy(ob, out_hbm.at[row0 + c], so)

  def wait_out():
    pltpu.make_async_copy(ob, out_hbm.at[0], so).wait()

  def compute(p):
    xp, ip, op = xb[p], idb[p], ob

    idx15 = jnp.full((L,), L - 1, jnp.int32)
    idx0 = jnp.full((L,), 0, jnp.int32)
    iota = lax.iota(jnp.int32, L)
    ck = [iota + k * L for k in range(K)]

    @plsc.parallel_loop(0, CHUNK, unroll=4)
    def _tok(t):
      idv = ip[pl.ds(t, L)]
      irow = jnp.take(idv, idx0)
      e = []
      for k in range(K):
        sl = pl.ds(k * L, L)
        e.append(xp[t, sl] + posv[t, sl]
                 + plsc.load_gather(tabv, [irow, ck[k]]))
      ssum = jnp.take(plsc.cumsum(e[0] + e[1] + e[2] + e[3]), idx15)
      qsum = jnp.take(plsc.cumsum(e[0] * e[0] + e[1] * e[1]
                                  + e[2] * e[2] + e[3] * e[3]), idx15)
      mean = ssum * (1.0 / D)
      var = qsum * (1.0 / D) - mean * mean
      r = _rsqrt16(var + EPS)
      for k in range(K):
        op[t, pl.ds(k * L, L)] = (e[k] - mean) * r * g[k] + bt[k]

  def stage(c, p, *, first=False, no_prefetch=False):
    wait_x(p)
    wait_ids(p)
    if not first:
      wait_out()
    compute(p)
    issue_out(c)
    if not no_prefetch:
      issue_ids(c + 2, p)
      issue_x(c + 2, p)

  issue_ids(0, 0)
  issue_x(0, 0)
  issue_ids(1, 1)
  issue_x(1, 1)

  stage(0, 0, first=True)
  stage(1, 1)

  @pl.loop(1, ROWS_W // 2 - 1)
  def _pair(cc):
    stage(2 * cc, 0)
    stage(2 * cc + 1, 1)

  stage(ROWS_W - 2, 0, no_prefetch=True)
  stage(ROWS_W - 1, 1, no_prefetch=True)
  wait_out()


@jax.jit
def kernel(input_tensor, label_ids, pos_table, label_table, ln_gamma, ln_beta):
  ids = label_ids.reshape(N).astype(jnp.int32)
  mesh = plsc.VectorSubcoreMesh(core_axis_name="core",
                                subcore_axis_name="subcore")
  cp = pltpu.CompilerParams(needs_layout_passes=False)
  run = pl.kernel(
      _body,
      out_type=jax.ShapeDtypeStruct((B, S, D), jnp.float32),
      mesh=mesh,
      scratch_types=[
          pltpu.VMEM((CHUNK, D), jnp.float32),
          pltpu.VMEM((CHUNK, D), jnp.float32),
          pltpu.VMEM((CHUNK, D), jnp.float32),
          pltpu.VMEM((IDPAD,), jnp.int32),
          pltpu.VMEM((IDPAD,), jnp.int32),
          pltpu.VMEM((S, D), jnp.float32),
          pltpu.VMEM((NLAB, D), jnp.float32),
          pltpu.VMEM((D,), jnp.float32),
          pltpu.VMEM((D,), jnp.float32),
          pltpu.SemaphoreType.DMA,
          pltpu.SemaphoreType.DMA,
          pltpu.SemaphoreType.DMA,
          pltpu.SemaphoreType.DMA,
          pltpu.SemaphoreType.DMA,
      ],
      compiler_params=cp,
  )
  return run(input_tensor, ids, pos_table, label_table, ln_gamma, ln_beta)


# speedup vs baseline: 2.5993x; 1.0556x over previous
import jax
import jax.numpy as jnp
from jax import lax
from jax.experimental import pallas as pl
from jax.experimental.pallas import tpu as pltpu
from jax.experimental.pallas import tpu_sc as plsc

B, S, D = 4096, 200, 64
N = B * S
EPS = 1e-12
NC, NS = 2, 16
NW = NC * NS
BC = B // NW
L = 16
G = BC // L
NLAB = 26
SP = S + 8


def _rsqrt16(v):
  i = plsc.bitcast(v, jnp.int32)
  i = jnp.int32(0x5F3759DF) - lax.shift_right_arithmetic(i, 1)
  r = plsc.bitcast(i, jnp.float32)
  for _ in range(3):
    r = r * (1.5 - 0.5 * v * r * r)
  return r


def _body(x_hbm, ids_hbm, pos_hbm, lab_hbm, gam_hbm, bet_hbm, out_hbm,
          xb0, xb1, ob, eb, idb0, idb1, ptv, tabf, gbf, bbf, gv, bv,
          sx0, sx1, si0, si1, so):
  xb = (xb0, xb1)
  idb = (idb0, idb1)
  sx = (sx0, sx1)
  si = (si0, si1)

  wid = lax.axis_index("subcore") * NC + lax.axis_index("core")
  wb = wid * BC

  pltpu.sync_copy(pos_hbm, ptv)
  pltpu.sync_copy(lab_hbm, tabf)
  pltpu.sync_copy(gam_hbm, gv)
  pltpu.sync_copy(bet_hbm, bv)

  for k in range(D // L):
    pg = gv[pl.ds(k * L, L)]
    pb = bv[pl.ds(k * L, L)]
    for j in range(L):
      jidx = jnp.full((L,), j, jnp.int32)
      gbf[pl.ds((k * L + j) * L, L)] = jnp.take(pg, jidx)
      bbf[pl.ds((k * L + j) * L, L)] = jnp.take(pb, jidx)

  def issue_ids(s, p):
    pltpu.async_copy(ids_hbm.at[s, pl.ds(wb, BC)], idb[p], si[p])

  def wait_ids(p):
    pltpu.make_async_copy(ids_hbm.at[0, pl.ds(0, BC)], idb[p], si[p]).wait()

  def issue_x(s, p):
    pltpu.async_copy(x_hbm.at[s, :, pl.ds(wb, BC)], xb[p], sx[p])

  def wait_x(p):
    pltpu.make_async_copy(x_hbm.at[0, :, pl.ds(0, BC)], xb[p], sx[p]).wait()

  def issue_out(s):
    pltpu.async_copy(ob, out_hbm.at[s, :, pl.ds(wb, BC)], so)

  def wait_out():
    pltpu.make_async_copy(ob, out_hbm.at[0, :, pl.ds(0, BC)], so).wait()

  def compute(p, s):
    xp, ip = xb[p], idb[p]
    s16 = (s // L) * L
    sidx = jnp.broadcast_to(s - s16, (L,)).astype(jnp.int32)

    ib = []
    for g in range(G):
      ib.append(ip[pl.ds(g * L, L)] * D)

    zero = jnp.zeros((L,), jnp.float32)
    init = (zero,) * (2 * G)

    @plsc.parallel_loop(0, D, unroll=2, carry=init)
    def _p1(d, acc):
      pvec = ptv[d, pl.ds(s16, L)]
      psp = jnp.take(pvec, sidx)
      outs = []
      for g in range(G):
        sl = pl.ds(g * L, L)
        e = xp[d, sl] + plsc.load_gather(tabf, [ib[g] + d]) + psp
        eb[d, sl] = e
        outs.append(acc[2 * g] + e)
        outs.append(acc[2 * g + 1] + e * e)
      return tuple(outs)

    acc = _p1
    mean = []
    r = []
    for g in range(G):
      m = acc[2 * g] * (1.0 / D)
      var = acc[2 * g + 1] * (1.0 / D) - m * m
      mean.append(m)
      r.append(_rsqrt16(var + EPS))

    @plsc.parallel_loop(0, D, unroll=2)
    def _p2(d):
      gg = gbf[pl.ds(d * L, L)]
      bb = bbf[pl.ds(d * L, L)]
      for g in range(G):
        sl = pl.ds(g * L, L)
        ob[d, sl] = (eb[d, sl] - mean[g]) * r[g] * gg + bb

  def stage(s, p, *, first=False, no_prefetch=False):
    wait_x(p)
    wait_ids(p)
    if not first:
      wait_out()
    compute(p, s)
    issue_out(s)
    if not no_prefetch:
      issue_ids(s + 2, p)
      issue_x(s + 2, p)

  issue_ids(0, 0)
  issue_x(0, 0)
  issue_ids(1, 1)
  issue_x(1, 1)

  stage(0, 0, first=True)
  stage(1, 1)

  @pl.loop(1, S // 2 - 1)
  def _pair(cc):
    stage(2 * cc, 0)
    stage(2 * cc + 1, 1)

  stage(S - 2, 0, no_prefetch=True)
  stage(S - 1, 1, no_prefetch=True)
  wait_out()


@jax.jit
def kernel(input_tensor, label_ids, pos_table, label_table, ln_gamma, ln_beta):
  xt = jnp.transpose(input_tensor, (1, 2, 0))
  idt = jnp.transpose(label_ids.astype(jnp.int32), (1, 0))
  posT = jnp.pad(jnp.transpose(pos_table[:S], (1, 0)),
                 ((0, 0), (0, SP - S)))
  labf = label_table.reshape(NLAB * D)
  mesh = plsc.VectorSubcoreMesh(core_axis_name="core",
                                subcore_axis_name="subcore")
  cp = pltpu.CompilerParams(needs_layout_passes=False)
  run = pl.kernel(
      _body,
      out_type=jax.ShapeDtypeStruct((S, D, B), jnp.float32),
      mesh=mesh,
      scratch_types=[
          pltpu.VMEM((D, BC), jnp.float32),
          pltpu.VMEM((D, BC), jnp.float32),
          pltpu.VMEM((D, BC), jnp.float32),
          pltpu.VMEM((D, BC), jnp.float32),
          pltpu.VMEM((BC,), jnp.int32),
          pltpu.VMEM((BC,), jnp.int32),
          pltpu.VMEM((D, SP), jnp.float32),
          pltpu.VMEM((NLAB * D,), jnp.float32),
          pltpu.VMEM((D * L,), jnp.float32),
          pltpu.VMEM((D * L,), jnp.float32),
          pltpu.VMEM((D,), jnp.float32),
          pltpu.VMEM((D,), jnp.float32),
          pltpu.SemaphoreType.DMA,
          pltpu.SemaphoreType.DMA,
          pltpu.SemaphoreType.DMA,
          pltpu.SemaphoreType.DMA,
          pltpu.SemaphoreType.DMA,
      ],
      compiler_params=cp,
  )
  out = run(xt, idt, posT, labf, ln_gamma, ln_beta)
  return jnp.transpose(out, (2, 0, 1))
